# baseline (speedup 1.0000x reference)
import functools

import jax
import jax.numpy as jnp
from jax.experimental import pallas as pl
from jax.experimental.pallas import tpu as pltpu

N = 4096
E = 8192
B = 128
BLK_E = 128

_HI = jax.lax.Precision.HIGHEST


def _bf16(v):
    return v.astype(jnp.bfloat16)


def _mxu1p(a, b):
    return jnp.dot(_bf16(a), _bf16(b), preferred_element_type=jnp.float32)


def _prologue_body(x_ref, w_ref, b_ref, out_ref):
    out_ref[...] = jnp.maximum(_mxu1p(x_ref[...], w_ref[...]) + b_ref[...],
                               0.0)


def _prologue(x, wbig, bbig):
    return pl.pallas_call(
        _prologue_body,
        out_shape=jax.ShapeDtypeStruct((N, 32), jnp.float32),
    )(x, wbig, bbig)


def _nnconv_bf16_body(x_ref, ea_ref, src_ref, dst_ref, w1_ref, b1_ref,
                      w2_ref, b2_ref, root_ref, bias_ref, out_ref, h_s,
                      xs_s, msg_s, cnt_s, *, in_c, out_c, n_e, relu_out,
                      round_w):
    e = pl.program_id(0)
    f32 = jnp.float32

    @pl.when(e == 0)
    def _init():
        out_ref[...] = jnp.zeros_like(out_ref)
        cnt_s[...] = jnp.zeros_like(cnt_s)

    h_s[...] = jnp.maximum(
        _mxu1p(ea_ref[...], w1_ref[...]) + b1_ref[...], 0.0)
    src = src_ref[0]
    oh_src = (src == jax.lax.broadcasted_iota(
        jnp.int32, (BLK_E, N), 1)).astype(f32)
    xs_s[...] = jnp.dot(oh_src, x_ref[...],
                        preferred_element_type=f32, precision=_HI)

    wg = jnp.dot(_bf16(h_s[...]), w2_ref[...],
                 preferred_element_type=f32) + b2_ref[...]
    xs16 = _bf16(xs_s[...]).astype(f32) if round_w else xs_s[...]
    csz = 32 if in_c >= 32 else in_c
    acc = jnp.zeros((BLK_E, out_c), f32)
    for c in range(0, in_c, csz):
        wc = wg[:, c * out_c:(c + csz) * out_c]
        if round_w:
            wc = _bf16(wc).astype(f32)
        w3 = wc.reshape(BLK_E, csz, out_c)
        acc = acc + jnp.sum(w3 * xs16[:, c:c + csz][:, :, None], axis=1)
    msg_s[...] = acc

    dst = dst_ref[0]
    oh_dst = (dst == jax.lax.broadcasted_iota(
        jnp.int32, (BLK_E, N), 1)).astype(f32)
    out_ref[...] += jax.lax.dot_general(
        oh_dst, msg_s[...], (((0,), (0,)), ((), ())),
        preferred_element_type=f32, precision=_HI)
    cnt_s[...] += jax.lax.dot_general(
        oh_dst, jnp.ones((BLK_E, 1), f32),
        (((0,), (0,)), ((), ())), preferred_element_type=f32, precision=_HI)

    @pl.when(e == n_e - 1)
    def _finish():
        res = (out_ref[...] / jnp.maximum(cnt_s[...], 1.0)
               + _mxu1p(x_ref[...], root_ref[...]) + bias_ref[...])
        if relu_out:
            res = jnp.maximum(res, 0.0)
        out_ref[...] = res


def _nnconv(x, ea, src, dst, w1, b1, w2p, b2r, root, bias, in_c, out_c,
            relu_out, round_w):
    n_e = E // BLK_E
    body = functools.partial(_nnconv_bf16_body, in_c=in_c, out_c=out_c,
                             n_e=n_e, relu_out=relu_out, round_w=round_w)
    w2_spec = pl.BlockSpec((128, in_c * out_c), lambda e: (0, 0))
    b2_spec = pl.BlockSpec((1, in_c * out_c), lambda e: (0, 0))
    return pl.pallas_call(
        body,
        grid=(n_e,),
        in_specs=[
            pl.BlockSpec((N, in_c), lambda e: (0, 0)),
            pl.BlockSpec((BLK_E, 16), lambda e: (e, 0)),
            pl.BlockSpec((1, BLK_E, 1), lambda e: (e, 0, 0)),
            pl.BlockSpec((1, BLK_E, 1), lambda e: (e, 0, 0)),
            pl.BlockSpec((16, 128), lambda e: (0, 0)),
            pl.BlockSpec((1, 128), lambda e: (0, 0)),
            w2_spec,
            b2_spec,
            pl.BlockSpec((in_c, out_c), lambda e: (0, 0)),
            pl.BlockSpec((1, out_c), lambda e: (0, 0)),
        ],
        out_specs=pl.BlockSpec((N, out_c), lambda e: (0, 0)),
        out_shape=jax.ShapeDtypeStruct((N, out_c), jnp.float32),
        scratch_shapes=[
            pltpu.VMEM((BLK_E, 128), jnp.float32),
            pltpu.VMEM((BLK_E, in_c), jnp.float32),
            pltpu.VMEM((BLK_E, out_c), jnp.float32),
            pltpu.VMEM((N, 1), jnp.float32),
        ],
        compiler_params=pltpu.CompilerParams(
            vmem_limit_bytes=100 * 1024 * 1024),
    )(x, ea, src, dst, w1, b1, w2p, b2r, root, bias)


def _s2s_head_body(hx_ref, batch_ref, wih_ref, whh_ref, bi_ref,
                   fc2w_ref, fc2b_ref, fc3w_ref, fc3b_ref,
                   fc4w_ref, fc4b_ref, fc5w_ref, fc5b_ref, out_ref):
    f32 = jnp.float32
    hx = hx_ref[...]
    oh = (batch_ref[...] == jax.lax.broadcasted_iota(
        jnp.int32, (N, B), 1)).astype(f32)
    wih16 = _bf16(wih_ref[...])
    whh16 = _bf16(whh_ref[...])
    bi = bi_ref[...]

    q_star = jnp.zeros((B, 512), f32)
    h = jnp.zeros((B, 256), f32)
    cst = jnp.zeros((B, 256), f32)
    for _ in range(2):
        gates = (jax.lax.dot_general(_bf16(q_star), wih16,
                                     (((1,), (1,)), ((), ())),
                                     preferred_element_type=f32)
                 + jax.lax.dot_general(_bf16(h), whh16,
                                       (((1,), (1,)), ((), ())),
                                       preferred_element_type=f32) + bi)
        ii = jax.nn.sigmoid(gates[:, 0:256])
        ff = jax.nn.sigmoid(gates[:, 256:512])
        gg = jnp.tanh(gates[:, 512:768])
        oo = jax.nn.sigmoid(gates[:, 768:1024])
        cst = ff * cst + ii * gg
        h = oo * jnp.tanh(cst)
        qb = jnp.dot(oh, h, preferred_element_type=f32,
                     precision=_HI)
        e = jnp.sum(hx * qb, axis=1, keepdims=True)
        masked = oh * e + (oh - 1.0) * 1e30
        emax = jnp.max(masked, axis=0, keepdims=True)
        emax_n = jax.lax.dot_general(oh, emax, (((1,), (1,)), ((), ())),
                                     preferred_element_type=f32,
                                     precision=_HI)
        ex = jnp.exp(e - emax_n)
        denom = jnp.sum(oh * ex, axis=0, keepdims=True)
        denom_n = jax.lax.dot_general(oh, denom, (((1,), (1,)), ((), ())),
                                      preferred_element_type=f32,
                                      precision=_HI)
        a = ex / denom_n
        r = jax.lax.dot_general(oh, a * hx, (((0,), (0,)), ((), ())),
                                preferred_element_type=f32,
                                precision=_HI)
        q_star = jnp.concatenate([h, r], axis=1)

    o = jnp.maximum(_mxu1p(q_star, fc2w_ref[...]) + fc2b_ref[...], 0.0)
    o = jnp.maximum(_mxu1p(o, fc3w_ref[...]) + fc3b_ref[...], 0.0)
    o = jnp.maximum(_mxu1p(o, fc4w_ref[...]) + fc4b_ref[...], 0.0)
    out_ref[...] = _mxu1p(o, fc5w_ref[...]) + fc5b_ref[...]


def _s2s_head(hx, batch2d, wih, whh, bi, fc2w, fc2b, fc3w, fc3b, fc4w, fc4b,
              fc5w, fc5b):
    return pl.pallas_call(
        _s2s_head_body,
        out_shape=jax.ShapeDtypeStruct((B, 1), jnp.float32),
    )(hx, batch2d, wih, whh, bi, fc2w, fc2b, fc3w, fc3b, fc4w, fc4b,
      fc5w, fc5b)


@jax.jit
def kernel(x, edge_index, edge_attr, batch, a1_w, a1_b, m1_w, m1_b, en3_w1, en3_b1, en3_w2, en3_b2, root3, bias3, en4_w1, en4_b1, en4_w2, en4_b2, root4, bias4, en5_w1, en5_b1, en5_w2, en5_b2, root5, bias5, lstm_wih, lstm_whh, lstm_bih, lstm_bhh, fc2_w, fc2_b, fc3_w, fc3_b, fc4_w, fc4_b, fc5_w, fc5_b):
    f32 = jnp.float32
    wbig = jnp.zeros((32, 32), f32).at[26:32, 0:6].set(a1_w).at[0:26, 6:32].set(m1_w)
    bbig = jnp.concatenate([a1_b, m1_b])[None, :]
    src = edge_index[0].astype(jnp.int32).reshape(E // BLK_E, BLK_E, 1)
    dst = edge_index[1].astype(jnp.int32).reshape(E // BLK_E, BLK_E, 1)
    batch2d = batch.astype(jnp.int32).reshape(N, 1)

    hx = _prologue(x, wbig, bbig)
    hx = _nnconv(hx, edge_attr, src, dst, en3_w1, en3_b1[None, :],
                 en3_w2.astype(jnp.bfloat16), en3_b2[None, :],
                 root3, bias3[None, :], 32, 64, True, True)
    hx = _nnconv(hx, edge_attr, src, dst, en4_w1, en4_b1[None, :],
                 en4_w2.astype(jnp.bfloat16), en4_b2[None, :],
                 root4, bias4[None, :], 64, 128, True, True)
    hx = _nnconv(hx, edge_attr, src, dst, en5_w1, en5_b1[None, :],
                 en5_w2.astype(jnp.bfloat16), en5_b2[None, :],
                 root5, bias5[None, :], 128, 256, False, True)
    return _s2s_head(hx, batch2d, lstm_wih, lstm_whh,
                     (lstm_bih + lstm_bhh)[None, :],
                     fc2_w, fc2_b[None, :], fc3_w, fc3_b[None, :],
                     fc4_w, fc4_b[None, :], fc5_w, fc5_b[None, :])
